# baseline (speedup 1.0000x reference)
import functools

import jax
import jax.numpy as jnp
from jax import lax
from jax.experimental import pallas as pl
from jax.experimental.pallas import tpu as pltpu
from jax.experimental.pallas import tpu_sc as plsc

_INFO = plsc.get_sparse_core_info()
_NC, _NS = _INFO.num_cores, _INFO.num_subcores
_NW = _NC * _NS
_CHUNK_ROWS = 32
_TC_ROWS = 5120
_TC_BLOCK_ROWS = 1024


def _tc_copy_body(src_ref, dst_ref):
    dst_ref[...] = src_ref[...]


def _make_sc_copy(base_row, sc_rows, dim, dtype):
    rows_per_w = sc_rows // _NW
    n_chunks = rows_per_w // _CHUNK_ROWS
    mesh = plsc.VectorSubcoreMesh(core_axis_name="c", subcore_axis_name="s")

    @functools.partial(
        pl.kernel,
        mesh=mesh,
        out_type=jax.ShapeDtypeStruct((sc_rows, dim), dtype),
        scratch_types=[
            pltpu.VMEM((2, _CHUNK_ROWS, dim), dtype),
            pltpu.SemaphoreType.DMA((2,)),
            pltpu.SemaphoreType.DMA((2,)),
        ],
    )
    def sc_copy(w_hbm, out_hbm, buf, rsem, wsem):
        wid = lax.axis_index("s") * _NC + lax.axis_index("c")
        src_base = base_row + wid * rows_per_w
        dst_base = wid * rows_per_w

        def rd(i):
            return pltpu.make_async_copy(
                w_hbm.at[pl.ds(src_base + i * _CHUNK_ROWS, _CHUNK_ROWS)],
                buf.at[i % 2],
                rsem.at[i % 2],
            )

        def wr(i):
            return pltpu.make_async_copy(
                buf.at[i % 2],
                out_hbm.at[pl.ds(dst_base + i * _CHUNK_ROWS, _CHUNK_ROWS)],
                wsem.at[i % 2],
            )

        rd(0).start()
        for i in range(n_chunks):
            if i + 1 < n_chunks:
                if i >= 1:
                    wr(i - 1).wait()
                rd(i + 1).start()
            rd(i).wait()
            wr(i).start()
        wr(n_chunks - 1).wait()
        if n_chunks >= 2:
            wr(n_chunks - 2).wait()

    return sc_copy


def kernel(inputs, weight):
    bsz, seq_len = inputs.shape[:2]
    dim = weight.shape[1]
    tc_rows = _TC_ROWS
    sc_rows = seq_len - tc_rows

    tc_out = pl.pallas_call(
        _tc_copy_body,
        out_shape=jax.ShapeDtypeStruct((tc_rows, dim), weight.dtype),
        grid=(tc_rows // _TC_BLOCK_ROWS,),
        in_specs=[pl.BlockSpec((_TC_BLOCK_ROWS, dim), lambda i: (i, 0))],
        out_specs=pl.BlockSpec((_TC_BLOCK_ROWS, dim), lambda i: (i, 0)),
    )(weight)

    sc_out = _make_sc_copy(tc_rows, sc_rows, dim, weight.dtype)(weight)

    return jnp.concatenate([tc_out, sc_out], axis=0)


# speedup vs baseline: 2.4645x; 2.4645x over previous
import jax
import jax.numpy as jnp
from jax.experimental import pallas as pl
from jax.experimental.pallas import tpu as pltpu

_CHUNK_ROWS = 1024
_NBUF = 2


def _copy_body(src_ref, dst_ref, buf, rsem, wsem):
    n_chunks = src_ref.shape[0] // _CHUNK_ROWS

    def rd(i):
        return pltpu.make_async_copy(
            src_ref.at[pl.ds(i * _CHUNK_ROWS, _CHUNK_ROWS)],
            buf.at[i % _NBUF],
            rsem.at[i % _NBUF],
        )

    def wr(i):
        return pltpu.make_async_copy(
            buf.at[i % _NBUF],
            dst_ref.at[pl.ds(i * _CHUNK_ROWS, _CHUNK_ROWS)],
            wsem.at[i % _NBUF],
        )

    rd(0).start()
    for i in range(n_chunks):
        if i + 1 < n_chunks:
            if i + 1 >= _NBUF:
                wr(i + 1 - _NBUF).wait()
            rd(i + 1).start()
        rd(i).wait()
        wr(i).start()
    for i in range(max(0, n_chunks - _NBUF), n_chunks):
        wr(i).wait()


def kernel(inputs, weight):
    bsz, seq_len = inputs.shape[:2]
    dim = weight.shape[1]
    return pl.pallas_call(
        _copy_body,
        out_shape=jax.ShapeDtypeStruct((seq_len, dim), weight.dtype),
        in_specs=[pl.BlockSpec(memory_space=pl.ANY)],
        out_specs=pl.BlockSpec(memory_space=pl.ANY),
        scratch_shapes=[
            pltpu.VMEM((_NBUF, _CHUNK_ROWS, dim), jnp.float32),
            pltpu.SemaphoreType.DMA((_NBUF,)),
            pltpu.SemaphoreType.DMA((_NBUF,)),
        ],
    )(weight)


# speedup vs baseline: 2.8507x; 1.1567x over previous
import jax
import jax.numpy as jnp
from jax.experimental import pallas as pl
from jax.experimental.pallas import tpu as pltpu

_CHUNK_ROWS = 2048
_NBUF = 3


def _copy_body(src_ref, dst_ref, buf, rsem, wsem):
    n_chunks = src_ref.shape[0] // _CHUNK_ROWS

    def rd(i):
        return pltpu.make_async_copy(
            src_ref.at[pl.ds(i * _CHUNK_ROWS, _CHUNK_ROWS)],
            buf.at[i % _NBUF],
            rsem.at[i % _NBUF],
        )

    def wr(i):
        return pltpu.make_async_copy(
            buf.at[i % _NBUF],
            dst_ref.at[pl.ds(i * _CHUNK_ROWS, _CHUNK_ROWS)],
            wsem.at[i % _NBUF],
        )

    rd(0).start()
    for i in range(n_chunks):
        if i + 1 < n_chunks:
            if i + 1 >= _NBUF:
                wr(i + 1 - _NBUF).wait()
            rd(i + 1).start()
        rd(i).wait()
        wr(i).start()
    for i in range(max(0, n_chunks - _NBUF), n_chunks):
        wr(i).wait()


def kernel(inputs, weight):
    bsz, seq_len = inputs.shape[:2]
    dim = weight.shape[1]
    return pl.pallas_call(
        _copy_body,
        out_shape=jax.ShapeDtypeStruct((seq_len, dim), weight.dtype),
        in_specs=[pl.BlockSpec(memory_space=pl.ANY)],
        out_specs=pl.BlockSpec(memory_space=pl.ANY),
        scratch_shapes=[
            pltpu.VMEM((_NBUF, _CHUNK_ROWS, dim), jnp.float32),
            pltpu.SemaphoreType.DMA((_NBUF,)),
            pltpu.SemaphoreType.DMA((_NBUF,)),
        ],
    )(weight)
